# Initial kernel scaffold; baseline (speedup 1.0000x reference)
import jax
import jax.numpy as jnp
from jax.experimental import pallas as pl


def kernel(node_rep, edge_index, edge_attr, W1, g1, b1, W2, g2, b2, epsilon):
    raise NotImplementedError("write your pallas kernel here")


# speedup vs baseline: 8.4532x; 8.4532x over previous
import functools

import jax
import jax.numpy as jnp
from jax import lax
from jax.experimental import pallas as pl
from jax.experimental.pallas import tpu as pltpu
from jax.experimental.pallas import tpu_sc as plsc

N = 10000
E = 320000
D = 128
H = 2 * D

NC = 2
NS = 16
L = 16
NW = NC * NS
EPW = E // NW
K = 80
NCHUNK = EPW // K
RW = 80
NRCHUNK = N // RW
RT = -(-NRCHUNK // NS)


NBUF = 3


def _sc_body(src_hbm, dst_hbm, ea_hbm, nr_hbm, out_hbm,
             sidx, didx0, didx1, didx2, msg0, msg1, msg2, acc,
             sema, semg, semd):
    c = lax.axis_index("c")
    s = lax.axis_index("s")
    wid = s * NC + c
    zero = jnp.zeros((L,), jnp.float32)
    msgs = [msg0, msg1, msg2]
    didxs = [didx0, didx1, didx2]
    buf = msg0

    pltpu.sync_copy(src_hbm.at[wid], sidx)

    def zrow(r, _):
        for j in range(D // L):
            buf[r, pl.ds(j * L, L)] = zero
        return ()
    lax.fori_loop(0, RW, zrow, ())
    for t in range(RT):
        cid = s + NS * t

        @pl.when(cid < NRCHUNK)
        def _():
            pltpu.sync_copy(buf, acc.at[pl.ds(cid * RW, RW), :])
    plsc.subcore_barrier()

    base = wid * NCHUNK * K

    def t0(ci, slot):
        pltpu.async_copy(ea_hbm.at[pl.ds(base + ci * K, K), :], msgs[slot],
                         sema.at[slot])
        pltpu.async_copy(dst_hbm.at[wid, ci], didxs[slot], semd.at[slot])

    def t1(ci, slot):
        pltpu.make_async_copy(ea_hbm.at[pl.ds(base, K), :], msgs[slot],
                              sema.at[slot]).wait()
        pltpu.async_copy(nr_hbm.at[sidx.at[ci]], msgs[slot], semg.at[slot],
                         add=True)

    def t2(ci, slot):
        pltpu.make_async_copy(nr_hbm.at[sidx.at[ci]], msgs[slot],
                              semg.at[slot]).wait()
        m = msgs[slot]

        def rrow(r, _):
            for j in range(D // L):
                v = m[r, pl.ds(j * L, L)]
                m[r, pl.ds(j * L, L)] = jnp.maximum(v, 0.0)
            return ()
        lax.fori_loop(0, K, rrow, ())
        pltpu.make_async_copy(dst_hbm.at[wid, 0], didxs[slot],
                              semd.at[slot]).wait()
        pltpu.sync_copy(m, acc.at[didxs[slot]], add=True)

    t0(0, 0)
    t0(1, 1)
    t1(0, 0)

    def body(i, _):
        for k in range(NBUF):
            ci = i * NBUF + k

            @pl.when(ci + 2 < NCHUNK)
            def _():
                t0(ci + 2, (k + 2) % NBUF)

            @pl.when(ci + 1 < NCHUNK)
            def _():
                t1(ci + 1, (k + 1) % NBUF)
            t2(ci, k)
        return ()
    lax.fori_loop(0, NCHUNK // NBUF, body, ())
    for k in range(NCHUNK % NBUF):
        ci = (NCHUNK // NBUF) * NBUF + k

        @pl.when(ci + 2 < NCHUNK)
        def _():
            t0(ci + 2, (k + 2) % NBUF)

        @pl.when(ci + 1 < NCHUNK)
        def _():
            t1(ci + 1, (k + 1) % NBUF)
        t2(ci, k)
    plsc.subcore_barrier()

    for t in range(RT):
        cid = s + NS * t

        @pl.when(cid < NRCHUNK)
        def _():
            pltpu.sync_copy(acc.at[pl.ds(cid * RW, RW), :], buf)
            pltpu.sync_copy(buf, out_hbm.at[c, pl.ds(cid * RW, RW), :])


@functools.cache
def _sc_segment():
    return pl.kernel(
        _sc_body,
        out_type=jax.ShapeDtypeStruct((NC, N, D), jnp.float32),
        mesh=plsc.VectorSubcoreMesh(core_axis_name="c", subcore_axis_name="s",
                                    num_cores=NC, num_subcores=NS),
        scratch_types=[
            pltpu.VMEM((NCHUNK, K), jnp.int32),
            pltpu.VMEM((K,), jnp.int32),
            pltpu.VMEM((K,), jnp.int32),
            pltpu.VMEM((K,), jnp.int32),
            pltpu.VMEM((K, D), jnp.float32),
            pltpu.VMEM((K, D), jnp.float32),
            pltpu.VMEM((K, D), jnp.float32),
            pltpu.VMEM_SHARED((N, D), jnp.float32),
            pltpu.SemaphoreType.DMA((NBUF,)),
            pltpu.SemaphoreType.DMA((NBUF,)),
            pltpu.SemaphoreType.DMA((NBUF,)),
        ],
    )


def _mlp_body(parts_ref, nr_ref, w1_ref, g1_ref, b1_ref, w2_ref, g2_ref,
              b2_ref, eps_ref, out_ref):
    h = parts_ref[0] + parts_ref[1] + (1.0 + eps_ref[0]) * nr_ref[...]
    y = jnp.dot(h, w1_ref[...], preferred_element_type=jnp.float32)
    mu = jnp.mean(y, axis=0, keepdims=True)
    var = jnp.mean((y - mu) ** 2, axis=0, keepdims=True)
    y = jnp.maximum((y - mu) * lax.rsqrt(var + 1e-5) * g1_ref[...]
                    + b1_ref[...], 0.0)
    z = jnp.dot(y, w2_ref[...], preferred_element_type=jnp.float32)
    mu2 = jnp.mean(z, axis=0, keepdims=True)
    var2 = jnp.mean((z - mu2) ** 2, axis=0, keepdims=True)
    out_ref[...] = jnp.maximum((z - mu2) * lax.rsqrt(var2 + 1e-5) * g2_ref[...]
                               + b2_ref[...], 0.0)


_mlp = pl.pallas_call(
    _mlp_body,
    out_shape=jax.ShapeDtypeStruct((N, D), jnp.float32),
    in_specs=[pl.BlockSpec(memory_space=pltpu.VMEM)] * 8
    + [pl.BlockSpec(memory_space=pltpu.SMEM)],
)


def kernel(node_rep, edge_index, edge_attr, W1, g1, b1, W2, g2, b2, epsilon):
    src = edge_index[0].reshape(NW, NCHUNK, K)
    dst = edge_index[1].reshape(NW, NCHUNK, K)
    parts = _sc_segment()(src, dst, edge_attr, node_rep)
    return _mlp(parts, node_rep, W1, g1.reshape(1, H), b1.reshape(1, H),
                W2, g2.reshape(1, D), b2.reshape(1, D), epsilon)
